# Initial kernel scaffold; baseline (speedup 1.0000x reference)
import jax
import jax.numpy as jnp
from jax.experimental import pallas as pl


def kernel(input_ids, pos_embedding):
    raise NotImplementedError("write your pallas kernel here")


# speedup vs baseline: 1.4734x; 1.4734x over previous
import functools

import jax
import jax.numpy as jnp
from jax import lax
from jax.experimental import pallas as pl
from jax.experimental.pallas import tpu as pltpu
from jax.experimental.pallas import tpu_sc as plsc

D_MODEL = 2048
SEQ_LEN = 2048

NUM_CORES = 2
NUM_SUBCORES = 16
NUM_WORKERS = NUM_CORES * NUM_SUBCORES
ROWS_PER_WORKER = SEQ_LEN // NUM_WORKERS
CHUNK_ROWS = 32
NUM_CHUNKS = ROWS_PER_WORKER // CHUNK_ROWS

_mesh = plsc.VectorSubcoreMesh(
    core_axis_name="c", subcore_axis_name="s",
    num_cores=NUM_CORES, num_subcores=NUM_SUBCORES,
)


@functools.partial(
    pl.kernel,
    mesh=_mesh,
    out_type=jax.ShapeDtypeStruct((SEQ_LEN, D_MODEL), jnp.float32),
    scratch_types=[
        pltpu.VMEM((CHUNK_ROWS, D_MODEL), jnp.float32),
        pltpu.SemaphoreType.DMA,
        pltpu.SemaphoreType.DMA,
    ],
)
def _gather_rows(table_hbm, out_hbm, buf, in_sem, out_sem):
    wid = lax.axis_index("s") * NUM_CORES + lax.axis_index("c")
    base = wid * ROWS_PER_WORKER
    for i in range(NUM_CHUNKS):
        row0 = base + i * CHUNK_ROWS
        pltpu.make_async_copy(
            table_hbm.at[pl.ds(row0, CHUNK_ROWS)], buf, in_sem
        ).start()
        pltpu.make_async_copy(
            table_hbm.at[pl.ds(row0, CHUNK_ROWS)], buf, in_sem
        ).wait()
        pltpu.make_async_copy(
            buf, out_hbm.at[pl.ds(row0, CHUNK_ROWS)], out_sem
        ).start()
        pltpu.make_async_copy(
            buf, out_hbm.at[pl.ds(row0, CHUNK_ROWS)], out_sem
        ).wait()


def kernel(input_ids, pos_embedding):
    del input_ids
    out = _gather_rows(pos_embedding)
    return out[None]
